# baseline (speedup 1.0000x reference)
import jax
import jax.numpy as jnp
from jax.experimental import pallas as pl
from jax.experimental.pallas import tpu as pltpu

HIDDEN = 4096
NUM_EXPERTS = 8
LANES = 128
BLOCK_ROWS = 1024


def _router_body(x_ref, gw_ref, tkw_ref, tki_ref, aux_ref, me_acc, ce_acc):
    i = pl.program_id(0)
    nsteps = pl.num_programs(0)

    x = x_ref[...]
    gw = gw_ref[...]
    logits = jax.lax.dot_general(
        x, gw, (((1,), (0,)), ((), ())), preferred_element_type=jnp.float32
    )

    lane = jax.lax.broadcasted_iota(jnp.int32, logits.shape, 1)
    valid = lane < NUM_EXPERTS
    neg = jnp.float32(-jnp.inf)
    lm = jnp.where(valid, logits, neg)

    m1 = jnp.max(lm, axis=1, keepdims=True)
    i1 = jnp.min(jnp.where(lm == m1, lane, LANES), axis=1, keepdims=True)
    lm2 = jnp.where(lane == i1, neg, lm)
    m2 = jnp.max(lm2, axis=1, keepdims=True)
    i2 = jnp.min(jnp.where(lm2 == m2, lane, LANES), axis=1, keepdims=True)

    e21 = jnp.exp(m2 - m1)
    w2 = e21 / (1.0 + e21)
    w1 = 1.0 - w2
    tkw_ref[...] = jnp.concatenate([w1, w2], axis=1)
    tki_ref[...] = jnp.concatenate([i1, i2], axis=1)

    p = jnp.where(valid, jnp.exp(lm - m1), 0.0)
    denom = jnp.sum(p, axis=1, keepdims=True)
    gates = p / denom
    me_part = jnp.sum(gates, axis=0, keepdims=True)
    ce_part = jnp.sum((gates > 0).astype(jnp.float32), axis=0, keepdims=True)

    @pl.when(i == 0)
    def _():
        me_acc[...] = jnp.zeros_like(me_acc)
        ce_acc[...] = jnp.zeros_like(ce_acc)

    me_acc[...] += me_part
    ce_acc[...] += ce_part

    @pl.when(i == nsteps - 1)
    def _():
        n = jnp.float32(nsteps * BLOCK_ROWS)
        aux_ref[...] = (jnp.sum(me_acc[...] * ce_acc[...]) / (n * n)).reshape(1, 1)


def kernel(hidden_states, gate_w):
    b, s, h = hidden_states.shape
    n_tokens = b * s
    x = hidden_states.reshape(n_tokens, h)
    gw = jnp.zeros((h, LANES), jnp.float32).at[:, :NUM_EXPERTS].set(gate_w.T)

    nsteps = n_tokens // BLOCK_ROWS
    tkw, tki, aux = pl.pallas_call(
        _router_body,
        grid=(nsteps,),
        in_specs=[
            pl.BlockSpec((BLOCK_ROWS, h), lambda i: (i, 0)),
            pl.BlockSpec((h, LANES), lambda i: (0, 0)),
        ],
        out_specs=[
            pl.BlockSpec((BLOCK_ROWS, 2), lambda i: (i, 0)),
            pl.BlockSpec((BLOCK_ROWS, 2), lambda i: (i, 0)),
            pl.BlockSpec((1, 1), lambda i: (0, 0)),
        ],
        out_shape=[
            jax.ShapeDtypeStruct((n_tokens, 2), jnp.float32),
            jax.ShapeDtypeStruct((n_tokens, 2), jnp.int32),
            jax.ShapeDtypeStruct((1, 1), jnp.float32),
        ],
        scratch_shapes=[
            pltpu.VMEM((1, LANES), jnp.float32),
            pltpu.VMEM((1, LANES), jnp.float32),
        ],
    )(x, gw)
    return (tkw, tki, aux[0, 0])


# speedup vs baseline: 1.0000x; 1.0000x over previous
import jax
import jax.numpy as jnp
from jax.experimental import pallas as pl
from jax.experimental.pallas import tpu as pltpu

HIDDEN = 4096
NUM_EXPERTS = 8
LANES = 128
BLOCK_ROWS = 1024


def _router_body(x_ref, gw_ref, tkw_ref, tki_ref, aux_ref, me_acc, ce_acc):
    i = pl.program_id(0)
    nsteps = pl.num_programs(0)

    x = x_ref[...]
    gw = gw_ref[...]
    logits = jax.lax.dot_general(
        x, gw, (((1,), (0,)), ((), ())), preferred_element_type=jnp.float32
    )

    lane = jax.lax.broadcasted_iota(jnp.int32, logits.shape, 1)
    valid = lane < NUM_EXPERTS
    neg = jnp.float32(-jnp.inf)
    lm = jnp.where(valid, logits, neg)

    m1 = jnp.max(lm, axis=1, keepdims=True)
    i1 = jnp.min(jnp.where(lm == m1, lane, LANES), axis=1, keepdims=True)
    lm2 = jnp.where(lane == i1, neg, lm)
    m2 = jnp.max(lm2, axis=1, keepdims=True)
    i2 = jnp.min(jnp.where(lm2 == m2, lane, LANES), axis=1, keepdims=True)

    e21 = jnp.exp(m2 - m1)
    w2 = e21 / (1.0 + e21)
    w1 = 1.0 - w2
    lane2 = lane == 1
    tkw_ref[...] = jnp.where(lane2, w2, w1)
    tki_ref[...] = jnp.where(lane2, i2, i1)

    p = jnp.where(valid, jnp.exp(lm - m1), 0.0)
    denom = jnp.sum(p, axis=1, keepdims=True)
    gates = p / denom
    me_part = jnp.sum(gates, axis=0, keepdims=True)
    ce_part = jnp.sum((gates > 0).astype(jnp.float32), axis=0, keepdims=True)

    @pl.when(i == 0)
    def _():
        me_acc[...] = jnp.zeros_like(me_acc)
        ce_acc[...] = jnp.zeros_like(ce_acc)

    me_acc[...] += me_part
    ce_acc[...] += ce_part

    @pl.when(i == nsteps - 1)
    def _():
        n = jnp.float32(nsteps * BLOCK_ROWS)
        aux_ref[...] = (jnp.sum(me_acc[...] * ce_acc[...]) / (n * n)).reshape(1, 1)


def kernel(hidden_states, gate_w):
    b, s, h = hidden_states.shape
    n_tokens = b * s
    x = hidden_states.reshape(n_tokens, h)
    gw = jnp.zeros((h, LANES), jnp.float32).at[:, :NUM_EXPERTS].set(gate_w.T)

    nsteps = n_tokens // BLOCK_ROWS
    tkw, tki, aux = pl.pallas_call(
        _router_body,
        grid=(nsteps,),
        in_specs=[
            pl.BlockSpec((BLOCK_ROWS, h), lambda i: (i, 0)),
            pl.BlockSpec((h, LANES), lambda i: (0, 0)),
        ],
        out_specs=[
            pl.BlockSpec((BLOCK_ROWS, LANES), lambda i: (i, 0)),
            pl.BlockSpec((BLOCK_ROWS, LANES), lambda i: (i, 0)),
            pl.BlockSpec((1, 1), lambda i: (0, 0)),
        ],
        out_shape=[
            jax.ShapeDtypeStruct((n_tokens, LANES), jnp.float32),
            jax.ShapeDtypeStruct((n_tokens, LANES), jnp.int32),
            jax.ShapeDtypeStruct((1, 1), jnp.float32),
        ],
        scratch_shapes=[
            pltpu.VMEM((1, LANES), jnp.float32),
            pltpu.VMEM((1, LANES), jnp.float32),
        ],
    )(x, gw)
    return (tkw[:, :2], tki[:, :2], aux[0, 0])
